# baseline (device time: 105712 ns/iter reference)
import jax
import jax.numpy as jnp
from jax import lax
from jax.experimental import pallas as pl
from jax.experimental.pallas import tpu as pltpu

N_DEV = 4
N_SEG = 2


def kernel(A, B):
    m, k = A.shape
    _, n = B.shape
    m_blk = m // N_DEV
    m_half = m_blk // 2
    m_seg = m_half // N_SEG

    def body(a_ref, b_ref, out_ref, cw_ref, ccw_ref, a_cw_ref, a_ccw_ref,
             cp_sems, cw_send, cw_recv, ccw_send, ccw_recv):
        my = lax.axis_index("i")
        left = lax.rem(my + N_DEV - 1, N_DEV)
        right = lax.rem(my + 1, N_DEV)

        barrier_sem = pltpu.get_barrier_semaphore()
        for nbr in (left, right):
            pl.semaphore_signal(
                barrier_sem, inc=1,
                device_id=(nbr,), device_id_type=pl.DeviceIdType.MESH,
            )
        pl.semaphore_wait(barrier_sem, 2)

        def stage(c, row_off, a_stage, sem_idx):
            cp = pltpu.make_async_copy(
                a_ref.at[pl.ds(c * m_blk + row_off, m_half), :],
                a_stage, cp_sems.at[sem_idx],
            )
            cp.start()
            cp.wait()

        def seg_dot(a_stage, j):
            return jnp.dot(
                a_stage[pl.ds(j * m_seg, m_seg), :], b_ref[:, :],
                preferred_element_type=jnp.float32,
            )

        def full_dot(a_stage):
            return jnp.dot(a_stage[:, :], b_ref[:, :],
                           preferred_element_type=jnp.float32)

        def make_rdma(direction_ref, sems_send, sems_recv, dst, h, j):
            return pltpu.make_async_remote_copy(
                src_ref=direction_ref.at[h % 3, pl.ds(j * m_seg, m_seg), :],
                dst_ref=direction_ref.at[(h + 1) % 3,
                                         pl.ds(j * m_seg, m_seg), :],
                send_sem=sems_send.at[h, j],
                recv_sem=sems_recv.at[h, j],
                device_id=(dst,),
                device_id_type=pl.DeviceIdType.MESH,
            )

        cw_rdmas = {}
        ccw_rdmas = {}

        def send(h, j, cw):
            key = (h, j)
            if cw:
                cw_rdmas[key] = make_rdma(cw_ref, cw_send, cw_recv, right, h, j)
                cw_rdmas[key].start()
            else:
                ccw_rdmas[key] = make_rdma(ccw_ref, ccw_send, ccw_recv,
                                           left, h, j)
                ccw_rdmas[key].start()

        stage(lax.rem(my + N_DEV - 1, N_DEV), 0, a_cw_ref, 0)
        cw_ref[0, 0:m_seg, :] = seg_dot(a_cw_ref, 0).astype(jnp.bfloat16)
        send(0, 0, cw=True)
        stage(lax.rem(my + 1, N_DEV), m_half, a_ccw_ref, 1)
        ccw_ref[0, 0:m_seg, :] = seg_dot(a_ccw_ref, 0).astype(jnp.bfloat16)
        send(0, 0, cw=False)
        cw_ref[0, m_seg:m_half, :] = seg_dot(a_cw_ref, 1).astype(jnp.bfloat16)
        send(0, 1, cw=True)
        ccw_ref[0, m_seg:m_half, :] = seg_dot(a_ccw_ref, 1).astype(jnp.bfloat16)
        send(0, 1, cw=False)

        for h in range(N_DEV - 1):
            r = (h + 1) % 3
            c1 = lax.rem(my + 2 * N_DEV - 2 - h, N_DEV)
            c2 = lax.rem(my + 2 + h, N_DEV)
            stage(c1, 0, a_cw_ref, 0)
            p1 = full_dot(a_cw_ref)
            stage(c2, m_half, a_ccw_ref, 1)
            p2 = full_dot(a_ccw_ref)
            last = h == N_DEV - 2
            if not last:
                p1 = p1.astype(jnp.bfloat16)
                p2 = p2.astype(jnp.bfloat16)
            for j in range(N_SEG):
                rows = pl.ds(j * m_seg, m_seg)
                lo, hi = j * m_seg, (j + 1) * m_seg
                cw_rdmas[(h, j)].wait()
                if not last:
                    cw_ref[r, rows, :] = cw_ref[r, rows, :] + p1[lo:hi, :]
                    send(h + 1, j, cw=True)
                else:
                    out_ref[lo:hi, :] = (
                        cw_ref[r, rows, :].astype(jnp.float32) + p1[lo:hi, :]
                    )
                ccw_rdmas[(h, j)].wait()
                if not last:
                    ccw_ref[r, rows, :] = ccw_ref[r, rows, :] + p2[lo:hi, :]
                    send(h + 1, j, cw=False)
                else:
                    out_ref[m_half + lo:m_half + hi, :] = (
                        ccw_ref[r, rows, :].astype(jnp.float32) + p2[lo:hi, :]
                    )

    return pl.pallas_call(
        body,
        out_shape=jax.ShapeDtypeStruct((m_blk, n), jnp.float32),
        in_specs=[
            pl.BlockSpec(memory_space=pl.ANY),
            pl.BlockSpec(memory_space=pltpu.VMEM),
        ],
        out_specs=pl.BlockSpec(memory_space=pltpu.VMEM),
        scratch_shapes=[
            pltpu.VMEM((3, m_half, n), jnp.bfloat16),
            pltpu.VMEM((3, m_half, n), jnp.bfloat16),
            pltpu.VMEM((m_half, k), jnp.float32),
            pltpu.VMEM((m_half, k), jnp.float32),
            pltpu.SemaphoreType.DMA((2,)),
            pltpu.SemaphoreType.DMA((N_DEV - 1, N_SEG)),
            pltpu.SemaphoreType.DMA((N_DEV - 1, N_SEG)),
            pltpu.SemaphoreType.DMA((N_DEV - 1, N_SEG)),
            pltpu.SemaphoreType.DMA((N_DEV - 1, N_SEG)),
        ],
        compiler_params=pltpu.CompilerParams(
            collective_id=0, vmem_limit_bytes=63 * 1024 * 1024
        ),
    )(A, B)


# device time: 104931 ns/iter; 1.0074x vs baseline; 1.0074x over previous
import jax
import jax.numpy as jnp
from jax import lax
from jax.experimental import pallas as pl
from jax.experimental.pallas import tpu as pltpu

N_DEV = 4
N_SEG = 4


def kernel(A, B):
    m, k = A.shape
    _, n = B.shape
    m_blk = m // N_DEV
    m_half = m_blk // 2
    m_seg = m_half // N_SEG

    def body(a_ref, b_ref, out_ref, cw_ref, ccw_ref, a_cw_ref, a_ccw_ref,
             cp_sems, cw_send, cw_recv, ccw_send, ccw_recv):
        my = lax.axis_index("i")
        left = lax.rem(my + N_DEV - 1, N_DEV)
        right = lax.rem(my + 1, N_DEV)

        barrier_sem = pltpu.get_barrier_semaphore()
        for nbr in (left, right):
            pl.semaphore_signal(
                barrier_sem, inc=1,
                device_id=(nbr,), device_id_type=pl.DeviceIdType.MESH,
            )
        pl.semaphore_wait(barrier_sem, 2)

        def stage(c, row_off, a_stage, sem_idx):
            cp = pltpu.make_async_copy(
                a_ref.at[pl.ds(c * m_blk + row_off, m_half), :],
                a_stage, cp_sems.at[sem_idx],
            )
            cp.start()
            cp.wait()

        def seg_dot(a_stage, j):
            return jnp.dot(
                a_stage[pl.ds(j * m_seg, m_seg), :], b_ref[:, :],
                preferred_element_type=jnp.float32,
            )

        def full_dot(a_stage):
            return jnp.dot(a_stage[:, :], b_ref[:, :],
                           preferred_element_type=jnp.float32)

        def make_rdma(direction_ref, sems_send, sems_recv, dst, h, j):
            return pltpu.make_async_remote_copy(
                src_ref=direction_ref.at[h % 3, pl.ds(j * m_seg, m_seg), :],
                dst_ref=direction_ref.at[(h + 1) % 3,
                                         pl.ds(j * m_seg, m_seg), :],
                send_sem=sems_send.at[h, j],
                recv_sem=sems_recv.at[h, j],
                device_id=(dst,),
                device_id_type=pl.DeviceIdType.MESH,
            )

        cw_rdmas = {}
        ccw_rdmas = {}

        def send(h, j, cw):
            key = (h, j)
            if cw:
                cw_rdmas[key] = make_rdma(cw_ref, cw_send, cw_recv, right, h, j)
                cw_rdmas[key].start()
            else:
                ccw_rdmas[key] = make_rdma(ccw_ref, ccw_send, ccw_recv,
                                           left, h, j)
                ccw_rdmas[key].start()

        stage(lax.rem(my + N_DEV - 1, N_DEV), 0, a_cw_ref, 0)
        for j in range(N_SEG):
            rows = pl.ds(j * m_seg, m_seg)
            cw_ref[0, rows, :] = seg_dot(a_cw_ref, j).astype(jnp.bfloat16)
            send(0, j, cw=True)
            if j == 0:
                stage(lax.rem(my + 1, N_DEV), m_half, a_ccw_ref, 1)
            ccw_ref[0, rows, :] = seg_dot(a_ccw_ref, j).astype(jnp.bfloat16)
            send(0, j, cw=False)

        for h in range(N_DEV - 1):
            r = (h + 1) % 3
            c1 = lax.rem(my + 2 * N_DEV - 2 - h, N_DEV)
            c2 = lax.rem(my + 2 + h, N_DEV)
            stage(c1, 0, a_cw_ref, 0)
            p1 = full_dot(a_cw_ref)
            stage(c2, m_half, a_ccw_ref, 1)
            p2 = full_dot(a_ccw_ref)
            last = h == N_DEV - 2
            if not last:
                p1 = p1.astype(jnp.bfloat16)
                p2 = p2.astype(jnp.bfloat16)
            for j in range(N_SEG):
                rows = pl.ds(j * m_seg, m_seg)
                lo, hi = j * m_seg, (j + 1) * m_seg
                cw_rdmas[(h, j)].wait()
                if not last:
                    cw_ref[r, rows, :] = cw_ref[r, rows, :] + p1[lo:hi, :]
                    send(h + 1, j, cw=True)
                else:
                    out_ref[lo:hi, :] = (
                        cw_ref[r, rows, :].astype(jnp.float32) + p1[lo:hi, :]
                    )
                ccw_rdmas[(h, j)].wait()
                if not last:
                    ccw_ref[r, rows, :] = ccw_ref[r, rows, :] + p2[lo:hi, :]
                    send(h + 1, j, cw=False)
                else:
                    out_ref[m_half + lo:m_half + hi, :] = (
                        ccw_ref[r, rows, :].astype(jnp.float32) + p2[lo:hi, :]
                    )

    return pl.pallas_call(
        body,
        out_shape=jax.ShapeDtypeStruct((m_blk, n), jnp.float32),
        in_specs=[
            pl.BlockSpec(memory_space=pl.ANY),
            pl.BlockSpec(memory_space=pltpu.VMEM),
        ],
        out_specs=pl.BlockSpec(memory_space=pltpu.VMEM),
        scratch_shapes=[
            pltpu.VMEM((3, m_half, n), jnp.bfloat16),
            pltpu.VMEM((3, m_half, n), jnp.bfloat16),
            pltpu.VMEM((m_half, k), jnp.float32),
            pltpu.VMEM((m_half, k), jnp.float32),
            pltpu.SemaphoreType.DMA((2,)),
            pltpu.SemaphoreType.DMA((N_DEV - 1, N_SEG)),
            pltpu.SemaphoreType.DMA((N_DEV - 1, N_SEG)),
            pltpu.SemaphoreType.DMA((N_DEV - 1, N_SEG)),
            pltpu.SemaphoreType.DMA((N_DEV - 1, N_SEG)),
        ],
        compiler_params=pltpu.CompilerParams(
            collective_id=0, vmem_limit_bytes=63 * 1024 * 1024
        ),
    )(A, B)


# device time: 104841 ns/iter; 1.0083x vs baseline; 1.0009x over previous
import jax
import jax.numpy as jnp
from jax import lax
from jax.experimental import pallas as pl
from jax.experimental.pallas import tpu as pltpu

N_DEV = 4
N_SEG = 4


def kernel(A, B):
    m, k = A.shape
    _, n = B.shape
    m_blk = m // N_DEV
    m_half = m_blk // 2
    m_seg = m_half // N_SEG

    def body(a_ref, b_ref, out_ref, cw_ref, ccw_ref, a_cw_ref, a_ccw_ref,
             cp_sems, cw_send, cw_recv, ccw_send, ccw_recv):
        my = lax.axis_index("i")
        left = lax.rem(my + N_DEV - 1, N_DEV)
        right = lax.rem(my + 1, N_DEV)

        barrier_sem = pltpu.get_barrier_semaphore()
        for nbr in (left, right):
            pl.semaphore_signal(
                barrier_sem, inc=1,
                device_id=(nbr,), device_id_type=pl.DeviceIdType.MESH,
            )
        pl.semaphore_wait(barrier_sem, 2)

        def stage(c, row_off, a_stage, sem_idx):
            cp = pltpu.make_async_copy(
                a_ref.at[pl.ds(c * m_blk + row_off, m_half), :],
                a_stage, cp_sems.at[sem_idx],
            )
            cp.start()
            cp.wait()

        def seg_dot(a_stage, j):
            return jnp.dot(
                a_stage[pl.ds(j * m_seg, m_seg), :], b_ref[:, :],
                preferred_element_type=jnp.float32,
            )

        def full_dot(a_stage):
            return jnp.dot(a_stage[:, :], b_ref[:, :],
                           preferred_element_type=jnp.float32)

        def make_rdma(direction_ref, sems_send, sems_recv, dst, h, j):
            return pltpu.make_async_remote_copy(
                src_ref=direction_ref.at[h % 3, pl.ds(j * m_seg, m_seg), :],
                dst_ref=direction_ref.at[(h + 1) % 3,
                                         pl.ds(j * m_seg, m_seg), :],
                send_sem=sems_send.at[h, j],
                recv_sem=sems_recv.at[h, j],
                device_id=(dst,),
                device_id_type=pl.DeviceIdType.MESH,
            )

        cw_rdmas = {}
        ccw_rdmas = {}

        def send(h, j, cw):
            key = (h, j)
            if cw:
                cw_rdmas[key] = make_rdma(cw_ref, cw_send, cw_recv, right, h, j)
                cw_rdmas[key].start()
            else:
                ccw_rdmas[key] = make_rdma(ccw_ref, ccw_send, ccw_recv,
                                           left, h, j)
                ccw_rdmas[key].start()

        stage(lax.rem(my + N_DEV - 1, N_DEV), 0, a_cw_ref, 0)
        for j in range(N_SEG):
            rows = pl.ds(j * m_seg, m_seg)
            cw_ref[0, rows, :] = seg_dot(a_cw_ref, j).astype(jnp.bfloat16)
            send(0, j, cw=True)
            if j == 0:
                stage(lax.rem(my + 1, N_DEV), m_half, a_ccw_ref, 1)
            ccw_ref[0, rows, :] = seg_dot(a_ccw_ref, j).astype(jnp.bfloat16)
            send(0, j, cw=False)

        for h in range(N_DEV - 1):
            r = (h + 1) % 3
            c1 = lax.rem(my + 2 * N_DEV - 2 - h, N_DEV)
            c2 = lax.rem(my + 2 + h, N_DEV)
            stage(c1, 0, a_cw_ref, 0)
            stage(c2, m_half, a_ccw_ref, 1)
            last = h == N_DEV - 2
            for j in range(N_SEG):
                rows = pl.ds(j * m_seg, m_seg)
                lo, hi = j * m_seg, (j + 1) * m_seg
                p1j = seg_dot(a_cw_ref, j)
                if not last:
                    p1j = p1j.astype(jnp.bfloat16)
                cw_rdmas[(h, j)].wait()
                if not last:
                    cw_ref[r, rows, :] = cw_ref[r, rows, :] + p1j
                    send(h + 1, j, cw=True)
                else:
                    out_ref[lo:hi, :] = (
                        cw_ref[r, rows, :].astype(jnp.float32) + p1j
                    )
                p2j = seg_dot(a_ccw_ref, j)
                if not last:
                    p2j = p2j.astype(jnp.bfloat16)
                ccw_rdmas[(h, j)].wait()
                if not last:
                    ccw_ref[r, rows, :] = ccw_ref[r, rows, :] + p2j
                    send(h + 1, j, cw=False)
                else:
                    out_ref[m_half + lo:m_half + hi, :] = (
                        ccw_ref[r, rows, :].astype(jnp.float32) + p2j
                    )

    return pl.pallas_call(
        body,
        out_shape=jax.ShapeDtypeStruct((m_blk, n), jnp.float32),
        in_specs=[
            pl.BlockSpec(memory_space=pl.ANY),
            pl.BlockSpec(memory_space=pltpu.VMEM),
        ],
        out_specs=pl.BlockSpec(memory_space=pltpu.VMEM),
        scratch_shapes=[
            pltpu.VMEM((3, m_half, n), jnp.bfloat16),
            pltpu.VMEM((3, m_half, n), jnp.bfloat16),
            pltpu.VMEM((m_half, k), jnp.float32),
            pltpu.VMEM((m_half, k), jnp.float32),
            pltpu.SemaphoreType.DMA((2,)),
            pltpu.SemaphoreType.DMA((N_DEV - 1, N_SEG)),
            pltpu.SemaphoreType.DMA((N_DEV - 1, N_SEG)),
            pltpu.SemaphoreType.DMA((N_DEV - 1, N_SEG)),
            pltpu.SemaphoreType.DMA((N_DEV - 1, N_SEG)),
        ],
        compiler_params=pltpu.CompilerParams(
            collective_id=0, vmem_limit_bytes=63 * 1024 * 1024
        ),
    )(A, B)
